# Initial kernel scaffold; baseline (speedup 1.0000x reference)
import jax
import jax.numpy as jnp
from jax.experimental import pallas as pl


def kernel(x, edge_index, W1, b1, g1, be1, W2, b2, g2, be2, W3, b3):
    raise NotImplementedError("write your pallas kernel here")


# speedup vs baseline: 5.9408x; 5.9408x over previous
import math

import jax
import jax.numpy as jnp
from jax import lax
from jax.experimental import pallas as pl
from jax.experimental.pallas import tpu as pltpu
from jax.experimental.pallas import tpu_sc as plsc

_N = 10000
_D = 256
_E = 160000
_EPS = 1e-5

_NPAD = 10240
_EPAD = 163840
_BLK = 256
_NBLK = _NPAD // _BLK
_HALF = 128

_CH = 128
_TILES = 16
_CHUNKS = _EPAD // _TILES // _CH
_GRP = 2
_NGRP = _CHUNKS // _GRP
_CHUNKS_DEG = _EPAD // 32 // _CH
_RPT = _NPAD // _TILES

_BNSCALE = 1.0 / math.sqrt(1.0 + _EPS)


def _deg_body(dst_hbm, c0_hbm, c1_hbm, hist, zbuf, ones, dsti):
    c = lax.axis_index("c")
    s = lax.axis_index("s")
    wid = c * _TILES + s

    def _z(i, carry):
        zbuf[i] = jnp.zeros((16,), jnp.float32)
        return carry

    lax.fori_loop(0, _RPT, _z, 0)

    def _o(i, carry):
        ones[i] = jnp.full((16,), 1.0, jnp.float32)
        return carry

    lax.fori_loop(0, _CH, _o, 0)

    pltpu.sync_copy(dst_hbm.at[wid], dsti)
    rs = pl.ds(s * _RPT, _RPT)
    pltpu.sync_copy(zbuf, hist.at[rs])
    plsc.subcore_barrier()

    for j in range(_CHUNKS_DEG):
        pltpu.sync_copy(ones, hist.at[dsti.at[j]], add=True)
    plsc.subcore_barrier()

    @pl.when(c == 0)
    def _():
        pltpu.sync_copy(hist.at[rs], c0_hbm.at[rs])

    @pl.when(c == 1)
    def _():
        pltpu.sync_copy(hist.at[rs], c1_hbm.at[rs])


def _spmm_body(hp0, hp1, src_hbm, dst_hbm, r0, r1, acc, srcw, dstw,
               rows0, rows1, g0, g1, s0, s1):
    c = lax.axis_index("c")
    s = lax.axis_index("s")
    rows = (rows0, rows1)
    gsem = (g0, g1)
    ssem = (s0, s1)

    rs = pl.ds(s * _RPT, _RPT)

    def _per_core(hp, r):
        pltpu.sync_copy(hp.at[rs], acc.at[rs])
        plsc.subcore_barrier()

        def _grp(i, carry):
            pltpu.sync_copy(src_hbm.at[s, pl.ds(_GRP * i, _GRP)], srcw)
            pltpu.sync_copy(dst_hbm.at[s, pl.ds(_GRP * i, _GRP)], dstw)
            hs = [pltpu.async_copy(hp.at[srcw.at[b]], rows[b], gsem[b])
                  for b in range(_GRP)]
            hs2 = []
            for b in range(_GRP):
                hs[b].wait()
                hs2.append(
                    pltpu.async_copy(rows[b], acc.at[dstw.at[b]], ssem[b],
                                     add=True))
            for h in hs2:
                h.wait()
            return carry

        lax.fori_loop(0, _NGRP, _grp, 0)
        plsc.subcore_barrier()
        pltpu.sync_copy(acc.at[rs], r.at[rs])

    @pl.when(c == 0)
    def _():
        _per_core(hp0, r0)

    @pl.when(c == 1)
    def _():
        _per_core(hp1, r1)


def _sc_mesh():
    return plsc.VectorSubcoreMesh(core_axis_name="c", subcore_axis_name="s")


def _deg_call(dst_dg):
    f = pl.kernel(
        _deg_body,
        out_type=[jax.ShapeDtypeStruct((_NPAD, 16), jnp.float32)] * 2,
        mesh=_sc_mesh(),
        scratch_types=[
            pltpu.VMEM_SHARED((_NPAD, 16), jnp.float32),
            pltpu.VMEM((_RPT, 16), jnp.float32),
            pltpu.VMEM((_CH, 16), jnp.float32),
            pltpu.VMEM((_CHUNKS_DEG, _CH), jnp.int32),
        ],
    )
    return f(dst_dg)


def _spmm_call(hp0, hp1, src_sp, dst_sp):
    f = pl.kernel(
        _spmm_body,
        out_type=[jax.ShapeDtypeStruct((_NPAD, _HALF), jnp.float32)] * 2,
        mesh=_sc_mesh(),
        scratch_types=(
            [pltpu.VMEM_SHARED((_NPAD, _HALF), jnp.float32),
             pltpu.VMEM((_GRP, _CH), jnp.int32),
             pltpu.VMEM((_GRP, _CH), jnp.int32)]
            + [pltpu.VMEM((_CH, _HALF), jnp.float32)] * _GRP
            + [pltpu.SemaphoreType.DMA] * (2 * _GRP)
        ),
    )
    return f(hp0, hp1, src_sp, dst_sp)


def _dinv(i, c0_ref, c1_ref):
    deg = c0_ref[:, 0:1] + c1_ref[:, 0:1] + 1.0
    row = i * _BLK + lax.broadcasted_iota(jnp.int32, (_BLK, 1), 0)
    return jnp.where(row < _N, lax.rsqrt(deg), 0.0)


def _tc1(x_ref, w_ref, c0_ref, c1_ref, hp0_ref, hp1_ref):
    dinv = _dinv(pl.program_id(0), c0_ref, c1_ref)
    h = jnp.dot(x_ref[...], w_ref[...], preferred_element_type=jnp.float32)
    hp = h * dinv
    hp0_ref[...] = hp[:, :_HALF]
    hp1_ref[...] = hp[:, _HALF:]


def _tc2(r0_ref, r1_ref, c0_ref, c1_ref, b_ref, g_ref, be_ref, w_ref,
         hp0_ref, hp1_ref):
    dinv = _dinv(pl.program_id(0), c0_ref, c1_ref)
    r = jnp.concatenate([r0_ref[...], r1_ref[...]], axis=1)
    xb = (r * dinv + b_ref[...]) * _BNSCALE * g_ref[...] + be_ref[...]
    xb = jnp.maximum(xb, 0.0)
    h = jnp.dot(xb, w_ref[...], preferred_element_type=jnp.float32)
    hp = h * dinv
    hp0_ref[...] = hp[:, :_HALF]
    hp1_ref[...] = hp[:, _HALF:]


def _tc3(r0_ref, r1_ref, c0_ref, c1_ref, b_ref, out_ref):
    dinv = _dinv(pl.program_id(0), c0_ref, c1_ref)
    r = jnp.concatenate([r0_ref[...], r1_ref[...]], axis=1)
    out_ref[...] = r * dinv + b_ref[...]


_ROWSPEC = pl.BlockSpec((_BLK, _D), lambda i: (i, 0))
_HALFSPEC = pl.BlockSpec((_BLK, _HALF), lambda i: (i, 0))
_WSPEC = pl.BlockSpec((_D, _D), lambda i: (0, 0))
_CSPEC = pl.BlockSpec((_BLK, 16), lambda i: (i, 0))
_VSPEC = pl.BlockSpec((1, _D), lambda i: (0, 0))
_HPSHAPE = [jax.ShapeDtypeStruct((_NPAD, _HALF), jnp.float32)] * 2


def _tc1_call(x, W, c0, c1):
    return pl.pallas_call(
        _tc1,
        grid=(_NBLK,),
        in_specs=[_ROWSPEC, _WSPEC, _CSPEC, _CSPEC],
        out_specs=[_HALFSPEC, _HALFSPEC],
        out_shape=_HPSHAPE,
    )(x, W, c0, c1)


def _tc2_call(r0, r1, c0, c1, b, g, be, W):
    return pl.pallas_call(
        _tc2,
        grid=(_NBLK,),
        in_specs=[_HALFSPEC, _HALFSPEC, _CSPEC, _CSPEC,
                  _VSPEC, _VSPEC, _VSPEC, _WSPEC],
        out_specs=[_HALFSPEC, _HALFSPEC],
        out_shape=_HPSHAPE,
    )(r0, r1, c0, c1, b, g, be, W)


def _tc3_call(r0, r1, c0, c1, b):
    return pl.pallas_call(
        _tc3,
        grid=(_NBLK,),
        in_specs=[_HALFSPEC, _HALFSPEC, _CSPEC, _CSPEC, _VSPEC],
        out_specs=_ROWSPEC,
        out_shape=jax.ShapeDtypeStruct((_NPAD, _D), jnp.float32),
    )(r0, r1, c0, c1, b)


def kernel(x, edge_index, W1, b1, g1, be1, W2, b2, g2, be2, W3, b3):
    ei = edge_index.astype(jnp.int32)
    pad = jnp.full((_EPAD - _E,), _NPAD - 1, jnp.int32)
    src = jnp.concatenate([ei[0], pad])
    dst = jnp.concatenate([ei[1], pad])
    src_sp = src.reshape(_TILES, _CHUNKS, _CH)
    dst_sp = dst.reshape(_TILES, _CHUNKS, _CH)
    dst_dg = dst.reshape(32, _CHUNKS_DEG, _CH)
    x_pad = jnp.pad(x, ((0, _NPAD - _N), (0, 0)))
    b1r, g1r, be1r = b1.reshape(1, _D), g1.reshape(1, _D), be1.reshape(1, _D)
    b2r, g2r, be2r = b2.reshape(1, _D), g2.reshape(1, _D), be2.reshape(1, _D)
    b3r = b3.reshape(1, _D)

    c0, c1 = _deg_call(dst_dg)
    hp0, hp1 = _tc1_call(x_pad, W1, c0, c1)
    r0, r1 = _spmm_call(hp0, hp1, src_sp, dst_sp)
    hp0, hp1 = _tc2_call(r0, r1, c0, c1, b1r, g1r, be1r, W2)
    r0, r1 = _spmm_call(hp0, hp1, src_sp, dst_sp)
    hp0, hp1 = _tc2_call(r0, r1, c0, c1, b2r, g2r, be2r, W3)
    r0, r1 = _spmm_call(hp0, hp1, src_sp, dst_sp)
    out = _tc3_call(r0, r1, c0, c1, b3r)
    return out[:_N]


# speedup vs baseline: 13.3453x; 2.2464x over previous
import math

import jax
import jax.numpy as jnp
from jax import lax
from jax.experimental import pallas as pl
from jax.experimental.pallas import tpu as pltpu
from jax.experimental.pallas import tpu_sc as plsc

_N = 10000
_D = 256
_E = 160000
_EPS = 1e-5

_NPAD = 10240
_EPAD = 163840
_BLK = 256
_NBLK = _NPAD // _BLK
_HALF = 128

_CH = 128
_TILES = 16
_CHUNKS_DEG = _EPAD // 32 // _CH
_RPT = _NPAD // _TILES

_SCH = 128
_SW = 4
_SNWIN = 21
_SC = _SW * _SNWIN
_SEPT = _SC * _SCH
_EPAD_S = _SEPT * _TILES

_BNSCALE = 1.0 / math.sqrt(1.0 + _EPS)


def _deg_body(dst_hbm, c0_hbm, c1_hbm, hist, zbuf, ones, dsti):
    c = lax.axis_index("c")
    s = lax.axis_index("s")
    wid = c * _TILES + s

    def _z(i, carry):
        zbuf[i] = jnp.zeros((16,), jnp.float32)
        return carry

    lax.fori_loop(0, _RPT, _z, 0)

    def _o(i, carry):
        ones[i] = jnp.full((16,), 1.0, jnp.float32)
        return carry

    lax.fori_loop(0, _CH, _o, 0)

    pltpu.sync_copy(dst_hbm.at[wid], dsti)
    rs = pl.ds(s * _RPT, _RPT)
    pltpu.sync_copy(zbuf, hist.at[rs])
    plsc.subcore_barrier()

    for j in range(_CHUNKS_DEG):
        pltpu.sync_copy(ones, hist.at[dsti.at[j]], add=True)
    plsc.subcore_barrier()

    @pl.when(c == 0)
    def _():
        pltpu.sync_copy(hist.at[rs], c0_hbm.at[rs])

    @pl.when(c == 1)
    def _():
        pltpu.sync_copy(hist.at[rs], c1_hbm.at[rs])


def _spmm_body(hp0, hp1, src_hbm, dst_hbm, r0, r1, acc, srcv0, srcv1,
               dstv0, dstv1, rows0, rows1, isem, g0, g1, s0, s1):
    c = lax.axis_index("c")
    s = lax.axis_index("s")
    rows = (rows0, rows1)
    gsem = (g0, g1)
    ssem = (s0, s1)

    rs = pl.ds(s * _RPT, _RPT)

    def _wait_gather(hp, b):
        pltpu.make_async_copy(hp.at[pl.ds(0, _SCH)], rows[b], gsem[b]).wait()

    def _wait_scatter(b):
        pltpu.make_async_copy(rows[b], acc.at[pl.ds(0, _SCH)], ssem[b]).wait()

    slots = ((srcv0, dstv0), (srcv1, dstv1))

    def _window(hp, w, sl, first, nxt_pred):
        swin, dwin = slots[sl]
        nsrc, ndst = slots[1 - sl]

        def _stage_next():
            pltpu.async_copy(src_hbm.at[s, w + 1], nsrc, isem)
            pltpu.async_copy(dst_hbm.at[s, w + 1], ndst, isem)

        def _wait_stage_and_prime(b):
            pltpu.make_async_copy(src_hbm.at[s, w + 1], nsrc, isem).wait()
            pltpu.make_async_copy(dst_hbm.at[s, w + 1], ndst, isem).wait()
            pltpu.async_copy(hp.at[nsrc.at[0]], rows[1 - b], gsem[1 - b])

        for m in range(_SW):
            b = m % 2
            _wait_gather(hp, b)
            pltpu.async_copy(rows[b], acc.at[dwin.at[m]], ssem[b], add=True)
            if m == 0:
                if not first:
                    _wait_scatter(1 - b)
                if nxt_pred is None:
                    _stage_next()
                else:
                    pl.when(nxt_pred)(_stage_next)
            else:
                _wait_scatter(1 - b)
            if m < _SW - 1:
                pltpu.async_copy(hp.at[swin.at[m + 1]], rows[1 - b],
                                 gsem[1 - b])
            else:
                if nxt_pred is None:
                    _wait_stage_and_prime(b)
                else:
                    pl.when(nxt_pred)(lambda: _wait_stage_and_prime(b))

    def _per_core(hp, r):
        i1 = pltpu.async_copy(src_hbm.at[s, 0], srcv0, isem)
        i2 = pltpu.async_copy(dst_hbm.at[s, 0], dstv0, isem)
        pltpu.sync_copy(hp.at[rs], acc.at[rs])
        i1.wait()
        i2.wait()
        pltpu.async_copy(hp.at[srcv0.at[0]], rows[0], gsem[0])
        plsc.subcore_barrier()

        _window(hp, 0, 0, True, None)

        def _pair(w2, carry):
            _window(hp, 2 * w2 + 1, 1, False, None)
            _window(hp, 2 * w2 + 2, 0, False, w2 <= (_SNWIN - 5) // 2)
            return carry

        lax.fori_loop(0, (_SNWIN - 1) // 2, _pair, 0)
        _wait_scatter((_SC - 1) % 2)
        plsc.subcore_barrier()
        pltpu.sync_copy(acc.at[rs], r.at[rs])

    @pl.when(c == 0)
    def _():
        _per_core(hp0, r0)

    @pl.when(c == 1)
    def _():
        _per_core(hp1, r1)


def _sc_mesh():
    return plsc.VectorSubcoreMesh(core_axis_name="c", subcore_axis_name="s")


def _deg_call(dst_dg):
    f = pl.kernel(
        _deg_body,
        out_type=[jax.ShapeDtypeStruct((_NPAD, 16), jnp.float32)] * 2,
        mesh=_sc_mesh(),
        scratch_types=[
            pltpu.VMEM_SHARED((_NPAD, 16), jnp.float32),
            pltpu.VMEM((_RPT, 16), jnp.float32),
            pltpu.VMEM((_CH, 16), jnp.float32),
            pltpu.VMEM((_CHUNKS_DEG, _CH), jnp.int32),
        ],
    )
    return f(dst_dg)


def _spmm_call(hp0, hp1, src_sp, dst_sp):
    f = pl.kernel(
        _spmm_body,
        out_type=[jax.ShapeDtypeStruct((_NPAD, _HALF), jnp.float32)] * 2,
        mesh=_sc_mesh(),
        scratch_types=(
            [pltpu.VMEM_SHARED((_NPAD, _HALF), jnp.float32)]
            + [pltpu.VMEM((_SW, _SCH), jnp.int32)] * 4
            + [pltpu.VMEM((_SCH, _HALF), jnp.float32)] * 2
            + [pltpu.SemaphoreType.DMA] * 5
        ),
    )
    return f(hp0, hp1, src_sp, dst_sp)


def _dinv(i, c0_ref, c1_ref):
    deg = c0_ref[:, 0:1] + c1_ref[:, 0:1] + 1.0
    row = i * _BLK + lax.broadcasted_iota(jnp.int32, (_BLK, 1), 0)
    return jnp.where(row < _N, lax.rsqrt(deg), 0.0)


def _tc1(x_ref, w_ref, c0_ref, c1_ref, hp0_ref, hp1_ref):
    dinv = _dinv(pl.program_id(0), c0_ref, c1_ref)
    h = jnp.dot(x_ref[...], w_ref[...], preferred_element_type=jnp.float32)
    hp = h * dinv
    hp0_ref[...] = hp[:, :_HALF]
    hp1_ref[...] = hp[:, _HALF:]


def _tc2(r0_ref, r1_ref, c0_ref, c1_ref, b_ref, g_ref, be_ref, w_ref,
         hp0_ref, hp1_ref):
    dinv = _dinv(pl.program_id(0), c0_ref, c1_ref)
    r = jnp.concatenate([r0_ref[...], r1_ref[...]], axis=1)
    xb = (r * dinv + b_ref[...]) * _BNSCALE * g_ref[...] + be_ref[...]
    xb = jnp.maximum(xb, 0.0)
    h = jnp.dot(xb, w_ref[...], preferred_element_type=jnp.float32)
    hp = h * dinv
    hp0_ref[...] = hp[:, :_HALF]
    hp1_ref[...] = hp[:, _HALF:]


def _tc3(r0_ref, r1_ref, c0_ref, c1_ref, b_ref, out_ref):
    dinv = _dinv(pl.program_id(0), c0_ref, c1_ref)
    r = jnp.concatenate([r0_ref[...], r1_ref[...]], axis=1)
    out_ref[...] = r * dinv + b_ref[...]


_ROWSPEC = pl.BlockSpec((_BLK, _D), lambda i: (i, 0))
_HALFSPEC = pl.BlockSpec((_BLK, _HALF), lambda i: (i, 0))
_WSPEC = pl.BlockSpec((_D, _D), lambda i: (0, 0))
_CSPEC = pl.BlockSpec((_BLK, 16), lambda i: (i, 0))
_VSPEC = pl.BlockSpec((1, _D), lambda i: (0, 0))
_HPSHAPE = [jax.ShapeDtypeStruct((_NPAD, _HALF), jnp.float32)] * 2


def _tc1_call(x, W, c0, c1):
    return pl.pallas_call(
        _tc1,
        grid=(_NBLK,),
        in_specs=[_ROWSPEC, _WSPEC, _CSPEC, _CSPEC],
        out_specs=[_HALFSPEC, _HALFSPEC],
        out_shape=_HPSHAPE,
    )(x, W, c0, c1)


def _tc2_call(r0, r1, c0, c1, b, g, be, W):
    return pl.pallas_call(
        _tc2,
        grid=(_NBLK,),
        in_specs=[_HALFSPEC, _HALFSPEC, _CSPEC, _CSPEC,
                  _VSPEC, _VSPEC, _VSPEC, _WSPEC],
        out_specs=[_HALFSPEC, _HALFSPEC],
        out_shape=_HPSHAPE,
    )(r0, r1, c0, c1, b, g, be, W)


def _tc3_call(r0, r1, c0, c1, b):
    return pl.pallas_call(
        _tc3,
        grid=(_NBLK,),
        in_specs=[_HALFSPEC, _HALFSPEC, _CSPEC, _CSPEC, _VSPEC],
        out_specs=_ROWSPEC,
        out_shape=jax.ShapeDtypeStruct((_NPAD, _D), jnp.float32),
    )(r0, r1, c0, c1, b)


def kernel(x, edge_index, W1, b1, g1, be1, W2, b2, g2, be2, W3, b3):
    ei = edge_index.astype(jnp.int32)
    pad_s = _N + jnp.arange(_EPAD_S - _E, dtype=jnp.int32) % (_NPAD - _N)
    src_sp = jnp.concatenate([ei[0], pad_s]).reshape(_TILES, _SNWIN, _SW,
                                                     _SCH)
    dst_sp = jnp.concatenate([ei[1], pad_s]).reshape(_TILES, _SNWIN, _SW,
                                                     _SCH)
    pad_d = jnp.full((_EPAD - _E,), _NPAD - 1, jnp.int32)
    dst_dg = jnp.concatenate([ei[1], pad_d]).reshape(32, _CHUNKS_DEG, _CH)
    x_pad = jnp.pad(x, ((0, _NPAD - _N), (0, 0)))
    b1r, g1r, be1r = b1.reshape(1, _D), g1.reshape(1, _D), be1.reshape(1, _D)
    b2r, g2r, be2r = b2.reshape(1, _D), g2.reshape(1, _D), be2.reshape(1, _D)
    b3r = b3.reshape(1, _D)

    c0, c1 = _deg_call(dst_dg)
    hp0, hp1 = _tc1_call(x_pad, W1, c0, c1)
    r0, r1 = _spmm_call(hp0, hp1, src_sp, dst_sp)
    hp0, hp1 = _tc2_call(r0, r1, c0, c1, b1r, g1r, be1r, W2)
    r0, r1 = _spmm_call(hp0, hp1, src_sp, dst_sp)
    hp0, hp1 = _tc2_call(r0, r1, c0, c1, b2r, g2r, be2r, W3)
    r0, r1 = _spmm_call(hp0, hp1, src_sp, dst_sp)
    out = _tc3_call(r0, r1, c0, c1, b3r)
    return out[:_N]
